# Initial kernel scaffold; baseline (speedup 1.0000x reference)
import jax
import jax.numpy as jnp
from jax.experimental import pallas as pl


def kernel(indices, atom_embedding):
    raise NotImplementedError("write your pallas kernel here")


# speedup vs baseline: 3.5421x; 3.5421x over previous
import functools

import jax
import jax.numpy as jnp
from jax import lax
from jax.experimental import pallas as pl
from jax.experimental.pallas import tpu as pltpu
from jax.experimental.pallas import tpu_sc as plsc

NUM_ROWS = 100000
D = 64
BATCH = 4096
HIST = 200
TOTAL = BATCH * HIST
NUM_WORKERS = 32
PER_W = TOTAL // NUM_WORKERS
CHUNK = 128
NCHUNK = PER_W // CHUNK


def _sc_gather(idx_hbm, table_hbm):
    mesh = plsc.VectorSubcoreMesh(core_axis_name="c", subcore_axis_name="s")

    @functools.partial(
        pl.kernel,
        mesh=mesh,
        out_type=jax.ShapeDtypeStruct((TOTAL, D), jnp.float32),
        compiler_params=pltpu.CompilerParams(use_tc_tiling_on_sc=False),
        scratch_types=[
            pltpu.VMEM((NCHUNK, CHUNK), jnp.int32),
            pltpu.VMEM((CHUNK, D), jnp.float32),
            pltpu.SemaphoreType.DMA,
        ],
    )
    def k(idx_ref, table_ref, out_ref, idx_v, rows_v, sem):
        wid = lax.axis_index("s") * 2 + lax.axis_index("c")
        base = wid * PER_W
        pltpu.sync_copy(idx_ref.at[wid], idx_v)

        def body(j, carry):
            pltpu.async_copy(table_ref.at[idx_v.at[j]], rows_v, sem).wait()
            pltpu.sync_copy(rows_v, out_ref.at[pl.ds(base + j * CHUNK, CHUNK)])
            return carry

        lax.fori_loop(0, NCHUNK, body, 0)

    return k(idx_hbm, table_hbm)


def kernel(indices, atom_embedding):
    idx = indices.astype(jnp.int32).reshape(NUM_WORKERS, NCHUNK, CHUNK)
    out = _sc_gather(idx, atom_embedding)
    return out.reshape(BATCH, HIST, D)


# speedup vs baseline: 4.2407x; 1.1972x over previous
import functools

import jax
import jax.numpy as jnp
from jax import lax
from jax.experimental import pallas as pl
from jax.experimental.pallas import tpu as pltpu
from jax.experimental.pallas import tpu_sc as plsc

NUM_ROWS = 100000
D = 64
BATCH = 4096
HIST = 200
TOTAL = BATCH * HIST
NUM_WORKERS = 32
PER_W = TOTAL // NUM_WORKERS
CHUNK = 128
NCHUNK = PER_W // CHUNK
NBUF = 4
ROUNDS = NCHUNK // NBUF


def _sc_gather(idx_hbm, table_hbm):
    mesh = plsc.VectorSubcoreMesh(core_axis_name="c", subcore_axis_name="s")

    @functools.partial(
        pl.kernel,
        mesh=mesh,
        out_type=jax.ShapeDtypeStruct((TOTAL, D), jnp.float32),
        compiler_params=pltpu.CompilerParams(use_tc_tiling_on_sc=False),
        scratch_types=(
            [pltpu.VMEM((NCHUNK, CHUNK), jnp.int32)]
            + [pltpu.VMEM((CHUNK, D), jnp.float32) for _ in range(NBUF)]
            + [pltpu.SemaphoreType.DMA for _ in range(2 * NBUF)]
        ),
    )
    def k(idx_ref, table_ref, out_ref, idx_v, *bufs_and_sems):
        rows = bufs_and_sems[:NBUF]
        gsem = bufs_and_sems[NBUF:2 * NBUF]
        ssem = bufs_and_sems[2 * NBUF:]
        wid = lax.axis_index("s") * 2 + lax.axis_index("c")
        base = wid * PER_W
        pltpu.sync_copy(idx_ref.at[wid], idx_v)

        def gather_start(b, j):
            pltpu.async_copy(table_ref.at[idx_v.at[j]], rows[b], gsem[b])

        def gather_wait(b):
            pltpu.make_async_copy(table_ref.at[idx_v.at[0]], rows[b],
                                  gsem[b]).wait()

        def store_start(b, j):
            pltpu.async_copy(rows[b],
                             out_ref.at[pl.ds(base + j * CHUNK, CHUNK)],
                             ssem[b])

        def store_wait(b):
            pltpu.make_async_copy(rows[b],
                                  out_ref.at[pl.ds(base, CHUNK)],
                                  ssem[b]).wait()

        for b in range(NBUF):
            gather_start(b, b)

        def round_body(r, carry):
            for b in range(NBUF):
                gather_wait(b)
                store_start(b, r * NBUF + b)

            @pl.when(r < ROUNDS - 1)
            def _():
                for b in range(NBUF):
                    store_wait(b)
                    gather_start(b, (r + 1) * NBUF + b)

            return carry

        lax.fori_loop(0, ROUNDS, round_body, 0)
        for b in range(NBUF):
            store_wait(b)

    return k(idx_hbm, table_hbm)


def kernel(indices, atom_embedding):
    idx = indices.astype(jnp.int32).reshape(NUM_WORKERS, NCHUNK, CHUNK)
    out = _sc_gather(idx, atom_embedding)
    return out.reshape(BATCH, HIST, D)


# speedup vs baseline: 4.2501x; 1.0022x over previous
import functools

import jax
import jax.numpy as jnp
from jax import lax
from jax.experimental import pallas as pl
from jax.experimental.pallas import tpu as pltpu
from jax.experimental.pallas import tpu_sc as plsc

NUM_ROWS = 100000
D = 64
BATCH = 4096
HIST = 200
TOTAL = BATCH * HIST
NUM_WORKERS = 32
PER_W = TOTAL // NUM_WORKERS
CHUNK = 128
NCHUNK = PER_W // CHUNK
NBUF = 8
ROUNDS = NCHUNK // NBUF


def _sc_gather(idx_hbm, table_hbm):
    mesh = plsc.VectorSubcoreMesh(core_axis_name="c", subcore_axis_name="s")

    @functools.partial(
        pl.kernel,
        mesh=mesh,
        out_type=jax.ShapeDtypeStruct((TOTAL, D), jnp.float32),
        compiler_params=pltpu.CompilerParams(use_tc_tiling_on_sc=False),
        scratch_types=(
            [pltpu.VMEM((NCHUNK, CHUNK), jnp.int32)]
            + [pltpu.VMEM((CHUNK, D), jnp.float32) for _ in range(NBUF)]
            + [pltpu.SemaphoreType.DMA for _ in range(2 * NBUF)]
        ),
    )
    def k(idx_ref, table_ref, out_ref, idx_v, *bufs_and_sems):
        rows = bufs_and_sems[:NBUF]
        gsem = bufs_and_sems[NBUF:2 * NBUF]
        ssem = bufs_and_sems[2 * NBUF:]
        wid = lax.axis_index("s") * 2 + lax.axis_index("c")
        base = wid * PER_W
        pltpu.sync_copy(idx_ref.at[wid], idx_v)

        def gather_start(b, j):
            pltpu.async_copy(table_ref.at[idx_v.at[j]], rows[b], gsem[b])

        def gather_wait(b):
            pltpu.make_async_copy(table_ref.at[idx_v.at[0]], rows[b],
                                  gsem[b]).wait()

        def store_start(b, j):
            pltpu.async_copy(rows[b],
                             out_ref.at[pl.ds(base + j * CHUNK, CHUNK)],
                             ssem[b])

        def store_wait(b):
            pltpu.make_async_copy(rows[b],
                                  out_ref.at[pl.ds(base, CHUNK)],
                                  ssem[b]).wait()

        for b in range(NBUF):
            gather_start(b, b)

        def round_body(r, carry):
            for b in range(NBUF):
                gather_wait(b)
                store_start(b, r * NBUF + b)

            @pl.when(r < ROUNDS - 1)
            def _():
                for b in range(NBUF):
                    store_wait(b)
                    gather_start(b, (r + 1) * NBUF + b)

            return carry

        lax.fori_loop(0, ROUNDS, round_body, 0)
        for b in range(NBUF):
            store_wait(b)

    return k(idx_hbm, table_hbm)


def kernel(indices, atom_embedding):
    idx = indices.astype(jnp.int32).reshape(NUM_WORKERS, NCHUNK, CHUNK)
    out = _sc_gather(idx, atom_embedding)
    return out.reshape(BATCH, HIST, D)
